# baseline (device time: 119983 ns/iter reference)
import jax
import jax.numpy as jnp
from jax import lax
from jax.experimental import pallas as pl
from jax.experimental.pallas import tpu as pltpu

BLK = 64
KBLK = 64
NSLOT = 2
NRSLOT = 4


def kernel(x, W):
    m, d = x.shape
    v = W.shape[1]
    n_blk = m // BLK
    n_k = d // KBLK
    n_steps = n_blk + 1

    xb = x.astype(jnp.bfloat16)

    def body(x_ref, w_hbm, out_ref, wb_ref, stage_ref, local_ref, peer_ref,
             w_sems, send_sems, recv_sems):
        my_x = lax.axis_index("x")
        my_y = lax.axis_index("y")
        my_z = lax.axis_index("z")
        peer = (my_x, 1 - my_y, my_z)
        j = pl.program_id(0)

        barrier_sem = pltpu.get_barrier_semaphore()

        def w_cp(k, slot):
            return pltpu.make_async_copy(
                w_hbm.at[pl.ds(k * KBLK, KBLK), :],
                stage_ref.at[slot],
                w_sems.at[slot],
            )

        def chunk_rdma(b, slot):
            return pltpu.make_async_remote_copy(
                src_ref=local_ref.at[slot],
                dst_ref=peer_ref.at[b % NRSLOT],
                send_sem=send_sems.at[b],
                recv_sem=recv_sems.at[b],
                device_id=peer,
                device_id_type=pl.DeviceIdType.MESH,
            )

        @pl.when(j == 0)
        def _():
            w_cp(0, 0).start()
            pl.semaphore_signal(
                barrier_sem, inc=1, device_id=peer,
                device_id_type=pl.DeviceIdType.MESH,
            )
            pl.semaphore_wait(barrier_sem, 1)

            for k in range(n_k):
                if k + 1 < n_k:
                    w_cp(k + 1, (k + 1) % 2).start()
                w_cp(k, k % 2).wait()
                wb_ref[pl.ds(k * KBLK, KBLK), :] = (
                    stage_ref[k % 2].astype(jnp.bfloat16)
                )

        @pl.when(j < n_blk)
        def _():
            rows = pl.ds(j * BLK, BLK)
            local_ref[j % NSLOT] = jnp.dot(
                x_ref[rows, :], wb_ref[:, :],
                preferred_element_type=jnp.float32,
            ).astype(jnp.bfloat16)
            chunk_rdma(j, j % NSLOT).start()

        @pl.when(j >= 1)
        def _():
            i = j - 1
            rdma = chunk_rdma(i, 0)
            rdma.wait_send()
            rdma.wait_recv()

            mine = local_ref[i % NSLOT].astype(jnp.float32)
            e0 = jnp.exp(mine)
            e1 = jnp.exp(peer_ref[i % NRSLOT].astype(jnp.float32))
            s = (
                jnp.sum(e0, axis=1, keepdims=True)
                + jnp.sum(e1, axis=1, keepdims=True)
            )
            r = 1.0 / s
            out_ref[:, pl.ds(my_y * v, v)] = (e0 * r).astype(jnp.bfloat16)
            out_ref[:, pl.ds((1 - my_y) * v, v)] = (e1 * r).astype(jnp.bfloat16)

    return pl.pallas_call(
        body,
        grid=(n_steps,),
        out_shape=jax.ShapeDtypeStruct((m, 2 * v), jnp.bfloat16),
        in_specs=[
            pl.BlockSpec((m, d), lambda j: (0, 0)),
            pl.BlockSpec(memory_space=pl.ANY),
        ],
        out_specs=pl.BlockSpec(
            (BLK, 2 * v), lambda j: (jnp.maximum(j - 1, 0), 0)
        ),
        scratch_shapes=[
            pltpu.VMEM((d, v), jnp.bfloat16),
            pltpu.VMEM((2, KBLK, v), jnp.float32),
            pltpu.VMEM((NSLOT, BLK, v), jnp.bfloat16),
            pltpu.VMEM((NRSLOT, BLK, v), jnp.bfloat16),
            pltpu.SemaphoreType.DMA((2,)),
            pltpu.SemaphoreType.DMA((m // BLK,)),
            pltpu.SemaphoreType.DMA((m // BLK,)),
        ],
        compiler_params=pltpu.CompilerParams(collective_id=0),
    )(xb, W)


# device time: 115912 ns/iter; 1.0351x vs baseline; 1.0351x over previous
import jax
import jax.numpy as jnp
from jax import lax
from jax.experimental import pallas as pl
from jax.experimental.pallas import tpu as pltpu

BLK = 32
KBLK = 128
NSLOT = 2
NRSLOT = 6
LAG = 2


def kernel(x, W):
    m, d = x.shape
    v = W.shape[1]
    n_blk = m // BLK
    n_k = d // KBLK
    n_steps = n_blk + LAG

    xb = x.astype(jnp.bfloat16)

    def body(x_ref, w_hbm, out_ref, wb_ref, stage_ref, local_ref, peer_ref,
             w_sems, send_sems, recv_sems):
        my_x = lax.axis_index("x")
        my_y = lax.axis_index("y")
        my_z = lax.axis_index("z")
        peer = (my_x, 1 - my_y, my_z)
        j = pl.program_id(0)

        barrier_sem = pltpu.get_barrier_semaphore()

        def w_cp(k, slot):
            return pltpu.make_async_copy(
                w_hbm.at[pl.ds(k * KBLK, KBLK), :],
                stage_ref.at[slot],
                w_sems.at[slot],
            )

        def chunk_rdma(b, slot):
            return pltpu.make_async_remote_copy(
                src_ref=local_ref.at[slot],
                dst_ref=peer_ref.at[b % NRSLOT],
                send_sem=send_sems.at[b],
                recv_sem=recv_sems.at[b],
                device_id=peer,
                device_id_type=pl.DeviceIdType.MESH,
            )

        @pl.when(j == 0)
        def _():
            w_cp(0, 0).start()
            pl.semaphore_signal(
                barrier_sem, inc=1, device_id=peer,
                device_id_type=pl.DeviceIdType.MESH,
            )
            pl.semaphore_wait(barrier_sem, 1)

            for k in range(n_k):
                if k + 1 < n_k:
                    w_cp(k + 1, (k + 1) % 2).start()
                w_cp(k, k % 2).wait()
                wb_ref[pl.ds(k * KBLK, KBLK), :] = (
                    stage_ref[k % 2].astype(jnp.bfloat16)
                )

        @pl.when(j >= LAG)
        def _():
            i = j - LAG
            rdma = chunk_rdma(i, 0)
            rdma.wait_send()
            rdma.wait_recv()

            mine = local_ref[i % NSLOT].astype(jnp.float32)
            e0 = jnp.exp(mine)
            e1 = jnp.exp(peer_ref[i % NRSLOT].astype(jnp.float32))
            s = (
                jnp.sum(e0, axis=1, keepdims=True)
                + jnp.sum(e1, axis=1, keepdims=True)
            )
            r = 1.0 / s
            out_ref[:, pl.ds(my_y * v, v)] = (e0 * r).astype(jnp.bfloat16)
            out_ref[:, pl.ds((1 - my_y) * v, v)] = (e1 * r).astype(jnp.bfloat16)

        @pl.when(j < n_blk)
        def _():
            rows = pl.ds(j * BLK, BLK)
            local_ref[j % NSLOT] = jnp.dot(
                x_ref[rows, :], wb_ref[:, :],
                preferred_element_type=jnp.float32,
            ).astype(jnp.bfloat16)
            chunk_rdma(j, j % NSLOT).start()

    return pl.pallas_call(
        body,
        grid=(n_steps,),
        out_shape=jax.ShapeDtypeStruct((m, 2 * v), jnp.bfloat16),
        in_specs=[
            pl.BlockSpec((m, d), lambda j: (0, 0)),
            pl.BlockSpec(memory_space=pl.ANY),
        ],
        out_specs=pl.BlockSpec(
            (BLK, 2 * v), lambda j: (jnp.maximum(j - LAG, 0), 0)
        ),
        scratch_shapes=[
            pltpu.VMEM((d, v), jnp.bfloat16),
            pltpu.VMEM((2, KBLK, v), jnp.float32),
            pltpu.VMEM((NSLOT, BLK, v), jnp.bfloat16),
            pltpu.VMEM((NRSLOT, BLK, v), jnp.bfloat16),
            pltpu.SemaphoreType.DMA((2,)),
            pltpu.SemaphoreType.DMA((m // BLK,)),
            pltpu.SemaphoreType.DMA((m // BLK,)),
        ],
        compiler_params=pltpu.CompilerParams(collective_id=0),
    )(xb, W)
